# baseline (speedup 1.0000x reference)
import functools

import jax
import jax.numpy as jnp
from jax import lax
from jax.experimental import pallas as pl
from jax.experimental.pallas import tpu as pltpu
from jax.experimental.pallas import tpu_sc as plsc

R = 16384
J = 50
D = 64
NUM_WORKERS = 32
RB = 128
N_RB = R // RB
RB_PER_W = N_RB // NUM_WORKERS


def _build_transpose():
    V = 1000000
    N_FULL = 7808
    PER_W = N_FULL // NUM_WORKERS
    mesh = plsc.VectorSubcoreMesh(core_axis_name="c", subcore_axis_name="s")

    @functools.partial(
        pl.kernel,
        mesh=mesh,
        out_type=jax.ShapeDtypeStruct((V // 2, 128), jnp.float32),
        scratch_types=[
            pltpu.VMEM((2, D, 128), jnp.float32),
            pltpu.VMEM((2, D, 128), jnp.float32),
            pltpu.VMEM((D, D), jnp.float32),
            pltpu.SemaphoreType.DMA,
            pltpu.SemaphoreType.DMA,
        ],
        compiler_params=pltpu.CompilerParams(needs_layout_passes=False),
    )
    def transpose_kernel(tab_t_hbm, tail_t_hbm, out_hbm, blk_v, t_v, tail_v,
                         gsem, wsem):
        wid = lax.axis_index("s") * 2 + lax.axis_index("c")
        iota16 = lax.iota(jnp.int32, 16)
        u0 = wid * PER_W

        def start_read(u, b):
            pltpu.async_copy(
                tab_t_hbm.at[:, pl.ds(u * 128, 128)], blk_v.at[b], gsem
            )

        def wait_read(b):
            pltpu.make_async_copy(
                tab_t_hbm.at[:, pl.ds(0, 128)], blk_v.at[b], gsem
            ).wait()

        def start_write(u, b):
            pltpu.async_copy(
                t_v.at[b], out_hbm.at[pl.ds(u * 64, 64)], wsem
            )

        def wait_write(b):
            pltpu.make_async_copy(
                t_v.at[b], out_hbm.at[pl.ds(0, 64)], wsem
            ).wait()

        def transpose(src, b, n_r):
            def per_r(r, carry):
                for cg in range(8):
                    dvec = (cg % 4) * 16 + iota16
                    vsplat = jnp.full((16,), 0, jnp.int32) + (2 * r + (1 if cg >= 4 else 0))
                    v = plsc.load_gather(src, [dvec, vsplat])
                    t_v[b, r, pl.ds(cg * 16, 16)] = v
                return carry
            lax.fori_loop(0, n_r, per_r, 0)

        start_read(u0, 0)
        wait_read(0)
        start_read(u0 + 1, 1)
        transpose(blk_v.at[0], 0, D)
        start_write(u0, 0)
        wait_read(1)
        start_read(u0 + 2, 0)
        transpose(blk_v.at[1], 1, D)
        start_write(u0 + 1, 1)

        def body(i, carry):
            for u_ in range(2):
                t = 2 * i + u_
                b = u_
                wait_read(b)

                @pl.when(t < PER_W - 1)
                def _():
                    start_read(u0 + t + 1, 1 - b)

                wait_write(b)
                transpose(blk_v.at[b], b, D)
                start_write(u0 + t, b)
            return carry
        lax.fori_loop(1, PER_W // 2, body, 0)
        wait_write(0)
        wait_write(1)

        @pl.when(wid < 4)
        def _():
            u = N_FULL + wid
            start_read(u, 0)
            wait_read(0)
            transpose(blk_v.at[0], 0, D)
            start_write(u, 0)
            wait_write(0)

        @pl.when(wid == 4)
        def _():
            u = N_FULL + 4
            pltpu.sync_copy(tail_t_hbm, tail_v)
            transpose(tail_v, 0, 32)
            pltpu.async_copy(
                t_v.at[0, pl.ds(0, 32)], out_hbm.at[pl.ds(u * 64, 32)], wsem
            )
            pltpu.make_async_copy(
                t_v.at[0, pl.ds(0, 32)], out_hbm.at[pl.ds(0, 32)], wsem
            ).wait()

    return transpose_kernel


def _build_gather():
    mesh = plsc.VectorSubcoreMesh(core_axis_name="c", subcore_axis_name="s")

    @functools.partial(
        pl.kernel,
        mesh=mesh,
        out_type=jax.ShapeDtypeStruct((J, D, R), jnp.float32),
        scratch_types=[
            pltpu.VMEM((RB * J,), jnp.int32),
            pltpu.VMEM((J, RB), jnp.int32),
            pltpu.VMEM((J, RB), jnp.int32),
            pltpu.VMEM((2, RB, 128), jnp.float32),
            pltpu.VMEM((2, D, RB), jnp.float32),
            pltpu.SemaphoreType.DMA,
            pltpu.SemaphoreType.DMA,
        ],
        compiler_params=pltpu.CompilerParams(needs_layout_passes=False),
    )
    def gather_kernel(idx_hbm, table2_hbm, out_hbm,
                      slab_v, idxcol_v, off_v, rows_v, t_v, gsem, wsem):
        wid = lax.axis_index("s") * 2 + lax.axis_index("c")
        iota16 = lax.iota(jnp.int32, 16)

        def start_gather(j, b):
            pltpu.async_copy(table2_hbm.at[idxcol_v.at[j]], rows_v.at[b], gsem)

        def wait_gather(b):
            pltpu.make_async_copy(
                table2_hbm.at[idxcol_v.at[0]], rows_v.at[b], gsem
            ).wait()

        def start_write(j, rb, b):
            pltpu.async_copy(
                t_v.at[b], out_hbm.at[j, :, pl.ds(rb * RB, RB)], wsem
            )

        def wait_write(b):
            pltpu.make_async_copy(
                t_v.at[b], out_hbm.at[0, :, pl.ds(0, RB)], wsem
            ).wait()

        def transpose(j, b):
            def per_d(d, carry):
                for kg in range(8):
                    kvec = kg * 16 + iota16
                    offv = off_v[j, pl.ds(kg * 16, 16)]
                    v = plsc.load_gather(rows_v.at[b], [kvec, offv + d])
                    t_v[b, d, pl.ds(kg * 16, 16)] = v
                return carry
            lax.fori_loop(0, D, per_d, 0)

        def per_rb(i, carry):
            rb = wid * RB_PER_W + i
            pltpu.sync_copy(idx_hbm.at[pl.ds(rb * RB * J, RB * J)], slab_v)

            def extract_j(j, c):
                for kg in range(8):
                    av = (kg * 16 + iota16) * J + j
                    v = plsc.load_gather(slab_v, [av])
                    idxcol_v[j, pl.ds(kg * 16, 16)] = v >> 1
                    off_v[j, pl.ds(kg * 16, 16)] = (v & 1) << 6
                return c
            lax.fori_loop(0, J, extract_j, 0)

            start_gather(0, 0)
            wait_gather(0)
            start_gather(1, 1)
            transpose(0, 0)
            start_write(0, rb, 0)
            wait_gather(1)
            start_gather(2, 0)
            transpose(1, 1)
            start_write(1, rb, 1)

            def body(i2, c):
                for u in range(2):
                    j = 2 * i2 + u
                    b = u
                    wait_gather(b)

                    @pl.when(j < J - 1)
                    def _():
                        start_gather(j + 1, 1 - b)

                    wait_write(b)
                    transpose(j, b)
                    start_write(j, rb, b)
                return c
            lax.fori_loop(1, J // 2, body, 0)

            wait_write(0)
            wait_write(1)
            return carry

        lax.fori_loop(0, RB_PER_W, per_rb, 0)

    return gather_kernel


def kernel(x, table):
    idx = x.reshape(R * J).astype(jnp.int32)
    table2 = _build_transpose()(table.T, table[999936:].T)
    out = _build_gather()(idx, table2)
    return out.transpose(2, 0, 1)


# speedup vs baseline: 2.2693x; 2.2693x over previous
import functools

import jax
import jax.numpy as jnp
from jax import lax
from jax.experimental import pallas as pl
from jax.experimental.pallas import tpu as pltpu
from jax.experimental.pallas import tpu_sc as plsc

R = 16384
J = 50
D = 64
NUM_WORKERS = 32
RB = 128
N_RB = R // RB
RB_PER_W = N_RB // NUM_WORKERS


def _build_transpose():
    V = 1000000
    N_FULL = 7808
    PER_W = N_FULL // NUM_WORKERS
    mesh = plsc.VectorSubcoreMesh(core_axis_name="c", subcore_axis_name="s")

    @functools.partial(
        pl.kernel,
        mesh=mesh,
        out_type=jax.ShapeDtypeStruct((V // 2, 128), jnp.float32),
        scratch_types=[
            pltpu.VMEM((2, D, 128), jnp.float32),
            pltpu.VMEM((2, D, 128), jnp.float32),
            pltpu.VMEM((D, D), jnp.float32),
            pltpu.SemaphoreType.DMA,
            pltpu.SemaphoreType.DMA,
        ],
        compiler_params=pltpu.CompilerParams(needs_layout_passes=False),
    )
    def transpose_kernel(tab_t_hbm, tail_t_hbm, out_hbm, blk_v, t_v, tail_v,
                         gsem, wsem):
        wid = lax.axis_index("s") * 2 + lax.axis_index("c")
        iota16 = lax.iota(jnp.int32, 16)
        u0 = wid * PER_W

        def start_read(u, b):
            pltpu.async_copy(
                tab_t_hbm.at[:, pl.ds(u * 128, 128)], blk_v.at[b], gsem
            )

        def wait_read(b):
            pltpu.make_async_copy(
                tab_t_hbm.at[:, pl.ds(0, 128)], blk_v.at[b], gsem
            ).wait()

        def start_write(u, b):
            pltpu.async_copy(
                t_v.at[b], out_hbm.at[pl.ds(u * 64, 64)], wsem
            )

        def wait_write(b):
            pltpu.make_async_copy(
                t_v.at[b], out_hbm.at[pl.ds(0, 64)], wsem
            ).wait()

        def transpose(src, b, n_r):
            for cg in range(8):
                dvec = (cg % 4) * 16 + iota16
                delta = 1 if cg >= 4 else 0
                sl = pl.ds(cg * 16, 16)

                @plsc.parallel_loop(0, n_r, unroll=8)
                def _(r):
                    vsplat = jnp.full((16,), 0, jnp.int32) + (2 * r + delta)
                    v = plsc.load_gather(src, [dvec, vsplat])
                    t_v[b, r, sl] = v

        start_read(u0, 0)
        wait_read(0)
        start_read(u0 + 1, 1)
        transpose(blk_v.at[0], 0, D)
        start_write(u0, 0)
        wait_read(1)
        start_read(u0 + 2, 0)
        transpose(blk_v.at[1], 1, D)
        start_write(u0 + 1, 1)

        def body(i, carry):
            for u_ in range(2):
                t = 2 * i + u_
                b = u_
                wait_read(b)

                @pl.when(t < PER_W - 1)
                def _():
                    start_read(u0 + t + 1, 1 - b)

                wait_write(b)
                transpose(blk_v.at[b], b, D)
                start_write(u0 + t, b)
            return carry
        lax.fori_loop(1, PER_W // 2, body, 0)
        wait_write(0)
        wait_write(1)

        @pl.when(wid < 4)
        def _():
            u = N_FULL + wid
            start_read(u, 0)
            wait_read(0)
            transpose(blk_v.at[0], 0, D)
            start_write(u, 0)
            wait_write(0)

        @pl.when(wid == 4)
        def _():
            u = N_FULL + 4
            pltpu.sync_copy(tail_t_hbm, tail_v)
            transpose(tail_v, 0, 32)
            pltpu.async_copy(
                t_v.at[0, pl.ds(0, 32)], out_hbm.at[pl.ds(u * 64, 32)], wsem
            )
            pltpu.make_async_copy(
                t_v.at[0, pl.ds(0, 32)], out_hbm.at[pl.ds(0, 32)], wsem
            ).wait()

    return transpose_kernel


def _build_gather():
    mesh = plsc.VectorSubcoreMesh(core_axis_name="c", subcore_axis_name="s")

    @functools.partial(
        pl.kernel,
        mesh=mesh,
        out_type=jax.ShapeDtypeStruct((J, D, R), jnp.float32),
        scratch_types=[
            pltpu.VMEM((RB * J,), jnp.int32),
            pltpu.VMEM((J, RB), jnp.int32),
            pltpu.VMEM((J, RB), jnp.int32),
            pltpu.VMEM((2, RB, 128), jnp.float32),
            pltpu.VMEM((2, D, RB), jnp.float32),
            pltpu.SemaphoreType.DMA,
            pltpu.SemaphoreType.DMA,
        ],
        compiler_params=pltpu.CompilerParams(needs_layout_passes=False),
    )
    def gather_kernel(idx_hbm, table2_hbm, out_hbm,
                      slab_v, idxcol_v, off_v, rows_v, t_v, gsem, wsem):
        wid = lax.axis_index("s") * 2 + lax.axis_index("c")
        iota16 = lax.iota(jnp.int32, 16)

        def start_gather(j, b):
            pltpu.async_copy(table2_hbm.at[idxcol_v.at[j]], rows_v.at[b], gsem)

        def wait_gather(b):
            pltpu.make_async_copy(
                table2_hbm.at[idxcol_v.at[0]], rows_v.at[b], gsem
            ).wait()

        def start_write(j, rb, b):
            pltpu.async_copy(
                t_v.at[b], out_hbm.at[j, :, pl.ds(rb * RB, RB)], wsem
            )

        def wait_write(b):
            pltpu.make_async_copy(
                t_v.at[b], out_hbm.at[0, :, pl.ds(0, RB)], wsem
            ).wait()

        def transpose(j, b):
            for kg in range(8):
                kvec = kg * 16 + iota16
                offv = off_v[j, pl.ds(kg * 16, 16)]
                sl = pl.ds(kg * 16, 16)

                @plsc.parallel_loop(0, D, unroll=8)
                def _(d):
                    v = plsc.load_gather(rows_v.at[b], [kvec, offv + d])
                    t_v[b, d, sl] = v

        def per_rb(i, carry):
            rb = wid * RB_PER_W + i
            pltpu.sync_copy(idx_hbm.at[pl.ds(rb * RB * J, RB * J)], slab_v)

            def extract_j(j, c):
                for kg in range(8):
                    av = (kg * 16 + iota16) * J + j
                    v = plsc.load_gather(slab_v, [av])
                    idxcol_v[j, pl.ds(kg * 16, 16)] = v >> 1
                    off_v[j, pl.ds(kg * 16, 16)] = (v & 1) << 6
                return c
            lax.fori_loop(0, J, extract_j, 0)

            start_gather(0, 0)
            wait_gather(0)
            start_gather(1, 1)
            transpose(0, 0)
            start_write(0, rb, 0)
            wait_gather(1)
            start_gather(2, 0)
            transpose(1, 1)
            start_write(1, rb, 1)

            def body(i2, c):
                for u in range(2):
                    j = 2 * i2 + u
                    b = u
                    wait_gather(b)

                    @pl.when(j < J - 1)
                    def _():
                        start_gather(j + 1, 1 - b)

                    wait_write(b)
                    transpose(j, b)
                    start_write(j, rb, b)
                return c
            lax.fori_loop(1, J // 2, body, 0)

            wait_write(0)
            wait_write(1)
            return carry

        lax.fori_loop(0, RB_PER_W, per_rb, 0)

    return gather_kernel


def kernel(x, table):
    idx = x.reshape(R * J).astype(jnp.int32)
    table2 = _build_transpose()(table.T, table[999936:].T)
    out = _build_gather()(idx, table2)
    return out.transpose(2, 0, 1)
